# baseline (speedup 1.0000x reference)
import functools

import jax
import jax.numpy as jnp
from jax import lax
from jax.experimental import pallas as pl
from jax.experimental.pallas import tpu as pltpu
from jax.experimental.pallas import tpu_sc as plsc

_N_TOK = 1152
_DIM = 64
_K = 1024
_COMMIT_W = 0.25
_ENT_W = 0.1
_INV_T = 100.0
_EPS = 1e-05


def _vq_body(x_ref, cb_ref, vq_ref, commit_ref, cbl_ref, ent_ref, idx_ref,
             cbpad_ref):
    flat = x_ref[...]
    cb = cb_ref[...]
    cbpad_ref[...] = jnp.concatenate(
        [cb, jnp.zeros((_K, 128 - _DIM), jnp.float32)], axis=1)

    xc = lax.dot_general(flat, cb, (((1,), (1,)), ((), ())),
                         preferred_element_type=jnp.float32,
                         precision=lax.Precision.HIGHEST)
    x2 = jnp.sum(flat * flat, axis=1, keepdims=True)
    ones = jnp.ones((1, _DIM), jnp.float32)
    c2 = lax.dot_general(ones, cb * cb, (((1,), (1,)), ((), ())),
                         preferred_element_type=jnp.float32,
                         precision=lax.Precision.HIGHEST)
    d2 = jnp.maximum(x2 - 2.0 * xc + c2, 0.0)
    d = jnp.sqrt(d2)

    dmin = jnp.min(d, axis=1, keepdims=True)
    fiota = lax.broadcasted_iota(jnp.int32, (_N_TOK, _K), 1).astype(
        jnp.float32)
    masked = jnp.where(d == dmin, fiota, float(_K))
    fidx = jnp.min(masked, axis=1, keepdims=True)
    idx_ref[...] = fidx.astype(jnp.int32)

    mse = jnp.sum(dmin * dmin) / (_N_TOK * _DIM)
    codebook_loss = 0.5 * mse
    commit_loss = codebook_loss * _COMMIT_W

    zc = (dmin - d) * _INV_T
    e = jnp.exp(zc)
    s = jnp.sum(e, axis=1, keepdims=True)
    rs_ezc = jnp.sum(e * zc, axis=1, keepdims=True)
    inv_s = 1.0 / s
    sample_entropy = -jnp.sum(rs_ezc * inv_s - jnp.log(s)) / _N_TOK
    avg_p = jnp.sum(e * inv_s, axis=0, keepdims=True) / _N_TOK
    avg_entropy = -jnp.sum(avg_p * jnp.log(avg_p + _EPS))
    entropy_loss = (sample_entropy - avg_entropy) * _ENT_W

    vq_ref[...] = jnp.reshape(codebook_loss + commit_loss + entropy_loss,
                              (1, 1))
    commit_ref[...] = jnp.reshape(commit_loss, (1, 1))
    cbl_ref[...] = jnp.reshape(codebook_loss, (1, 1))
    ent_ref[...] = jnp.reshape(entropy_loss, (1, 1))


_SC_INFO = plsc.get_sparse_core_info()
_NC = _SC_INFO.num_cores
_NS = _SC_INFO.num_subcores
_NW_USED = 24
_ROWS_PER_W = _N_TOK // _NW_USED


@functools.partial(
    pl.kernel,
    mesh=plsc.VectorSubcoreMesh(core_axis_name="c", subcore_axis_name="s"),
    out_type=jax.ShapeDtypeStruct((_N_TOK, 128), jnp.float32),
    scratch_types=[
        pltpu.VMEM((_ROWS_PER_W,), jnp.int32),
        pltpu.VMEM((_ROWS_PER_W, 128), jnp.float32),
        pltpu.SemaphoreType.DMA,
    ],
)
def _sc_gather(table_hbm, idx_hbm, out_hbm, idx_v, rows_v, sem):
    wid = lax.axis_index("s") * _NC + lax.axis_index("c")

    @pl.when(wid < _NW_USED)
    def _():
        base = wid * _ROWS_PER_W
        pltpu.sync_copy(idx_hbm.at[pl.ds(base, _ROWS_PER_W)], idx_v)
        pltpu.async_copy(table_hbm.at[idx_v], rows_v, sem).wait()
        pltpu.sync_copy(rows_v, out_hbm.at[pl.ds(base, _ROWS_PER_W)])


def kernel(x, codebook):
    flat = x.reshape(_N_TOK, _DIM)
    scalar = jax.ShapeDtypeStruct((1, 1), jnp.float32)
    out_shape = (
        scalar, scalar, scalar, scalar,
        jax.ShapeDtypeStruct((_N_TOK, 1), jnp.int32),
        jax.ShapeDtypeStruct((_K, 128), jnp.float32),
    )
    vq, commit, cbl, ent, idx, cbpad = pl.pallas_call(
        _vq_body,
        out_shape=out_shape,
    )(flat, codebook)
    q = _sc_gather(cbpad, idx.reshape(_N_TOK))[:, :_DIM]
    return (q.reshape(x.shape),
            vq.reshape(()), commit.reshape(()), cbl.reshape(()),
            ent.reshape(()),
            idx.reshape(x.shape[:-1]))


# speedup vs baseline: 2.0093x; 2.0093x over previous
import jax
import jax.numpy as jnp
from jax import lax
from jax.experimental import pallas as pl

_N_TOK = 1152
_DIM = 64
_K = 1024
_COMMIT_W = 0.25
_ENT_W = 0.1
_INV_T = 100.0
_EPS = 1e-05


def _vq_body(x_ref, cb_ref, q_ref, vq_ref, commit_ref, cbl_ref, ent_ref,
             idx_ref):
    flat = x_ref[...].reshape(_N_TOK, _DIM)
    cb = cb_ref[...]

    xc = lax.dot_general(flat, cb, (((1,), (1,)), ((), ())),
                         preferred_element_type=jnp.float32,
                         precision=lax.Precision.HIGHEST)
    x2 = jnp.sum(flat * flat, axis=1, keepdims=True)
    ones = jnp.ones((1, _DIM), jnp.float32)
    c2 = lax.dot_general(ones, cb * cb, (((1,), (1,)), ((), ())),
                         preferred_element_type=jnp.float32,
                         precision=lax.Precision.HIGHEST)
    d2 = jnp.maximum(x2 - 2.0 * xc + c2, 0.0)
    d = jnp.sqrt(d2)

    dmin = jnp.min(d, axis=1, keepdims=True)
    fiota = lax.broadcasted_iota(jnp.int32, (_N_TOK, _K), 1).astype(
        jnp.float32)
    masked = jnp.where(d == dmin, fiota, float(_K))
    fidx = jnp.min(masked, axis=1, keepdims=True)
    idx_ref[...] = fidx.astype(jnp.int32)

    onehot = (masked == fidx).astype(jnp.float32)
    q = lax.dot_general(onehot, cb, (((1,), (0,)), ((), ())),
                        preferred_element_type=jnp.float32)
    q_ref[...] = (flat + (q - flat)).reshape(2, _N_TOK // 2, _DIM)

    mse = jnp.sum(dmin * dmin) / (_N_TOK * _DIM)
    codebook_loss = 0.5 * mse
    commit_loss = codebook_loss * _COMMIT_W

    zc = (dmin - d) * _INV_T
    e = jnp.exp(zc)
    s = jnp.sum(e, axis=1, keepdims=True)
    rs_ezc = jnp.sum(e * zc, axis=1, keepdims=True)
    inv_s = 1.0 / s
    sample_entropy = -jnp.sum(rs_ezc * inv_s - jnp.log(s)) / _N_TOK
    avg_p = jnp.sum(e * inv_s, axis=0, keepdims=True) / _N_TOK
    avg_entropy = -jnp.sum(avg_p * jnp.log(avg_p + _EPS))
    entropy_loss = (sample_entropy - avg_entropy) * _ENT_W

    vq_ref[...] = jnp.reshape(codebook_loss + commit_loss + entropy_loss,
                              (1, 1))
    commit_ref[...] = jnp.reshape(commit_loss, (1, 1))
    cbl_ref[...] = jnp.reshape(codebook_loss, (1, 1))
    ent_ref[...] = jnp.reshape(entropy_loss, (1, 1))


def kernel(x, codebook):
    scalar = jax.ShapeDtypeStruct((1, 1), jnp.float32)
    out_shape = (
        jax.ShapeDtypeStruct((2, _N_TOK // 2, _DIM), jnp.float32),
        scalar, scalar, scalar, scalar,
        jax.ShapeDtypeStruct((_N_TOK, 1), jnp.int32),
    )
    q, vq, commit, cbl, ent, idx = pl.pallas_call(
        _vq_body,
        out_shape=out_shape,
    )(x, codebook)
    return (q,
            vq.reshape(()), commit.reshape(()), cbl.reshape(()),
            ent.reshape(()),
            idx.reshape(x.shape[:-1]))


# speedup vs baseline: 2.3346x; 1.1619x over previous
import jax
import jax.numpy as jnp
from jax import lax
from jax.experimental import pallas as pl

_N_TOK = 1152
_DIM = 64
_K = 1024
_COMMIT_W = 0.25
_ENT_W = 0.1
_INV_T = 100.0
_EPS = 1e-05


def _vq_body(x_ref, cb_ref, q_ref, vq_ref, commit_ref, cbl_ref, ent_ref,
             idx_ref):
    flat = x_ref[...].reshape(_N_TOK, _DIM)
    cb = cb_ref[...]

    x2 = jnp.sum(flat * flat, axis=1, keepdims=True)
    c2 = jnp.sum(cb * cb, axis=1, keepdims=True)
    a_aug = jnp.concatenate(
        [flat * -2.0, x2, jnp.ones((_N_TOK, 1), jnp.float32)], axis=1)
    b_aug = jnp.concatenate(
        [cb, jnp.ones((_K, 1), jnp.float32), c2], axis=1)
    d2 = lax.dot_general(a_aug, b_aug, (((1,), (1,)), ((), ())),
                         preferred_element_type=jnp.float32,
                         precision=lax.Precision.HIGHEST)
    d2 = jnp.maximum(d2, 0.0)
    d = jnp.sqrt(d2)

    dmin = jnp.min(d, axis=1, keepdims=True)
    fiota = lax.broadcasted_iota(jnp.int32, (_N_TOK, _K), 1).astype(
        jnp.float32)
    masked = jnp.where(d == dmin, fiota, float(_K))
    fidx = jnp.min(masked, axis=1, keepdims=True)
    idx_ref[...] = jnp.reshape(fidx.astype(jnp.int32), (2, _N_TOK // 2))

    onehot = (masked == fidx).astype(jnp.float32)
    q = lax.dot_general(onehot, cb, (((1,), (0,)), ((), ())),
                        preferred_element_type=jnp.float32)
    q_ref[...] = (flat + (q - flat)).reshape(2, _N_TOK // 2, _DIM)

    mse = jnp.sum(dmin * dmin) / (_N_TOK * _DIM)
    codebook_loss = 0.5 * mse
    commit_loss = codebook_loss * _COMMIT_W

    zc = (dmin - d) * _INV_T
    e = jnp.exp(zc)
    s = jnp.sum(e, axis=1, keepdims=True)
    rs_ezc = jnp.sum(e * zc, axis=1, keepdims=True)
    inv_s = 1.0 / s
    sample_entropy = -jnp.sum(rs_ezc * inv_s - jnp.log(s)) / _N_TOK
    avg_p = jnp.sum(e * inv_s, axis=0, keepdims=True) / _N_TOK
    avg_entropy = -jnp.sum(avg_p * jnp.log(avg_p + _EPS))
    entropy_loss = (sample_entropy - avg_entropy) * _ENT_W

    vq_ref[...] = jnp.reshape(codebook_loss + commit_loss + entropy_loss,
                              (1, 1))
    commit_ref[...] = jnp.reshape(commit_loss, (1, 1))
    cbl_ref[...] = jnp.reshape(codebook_loss, (1, 1))
    ent_ref[...] = jnp.reshape(entropy_loss, (1, 1))


def kernel(x, codebook):
    scalar = jax.ShapeDtypeStruct((1, 1), jnp.float32)
    out_shape = (
        jax.ShapeDtypeStruct((2, _N_TOK // 2, _DIM), jnp.float32),
        scalar, scalar, scalar, scalar,
        jax.ShapeDtypeStruct((2, _N_TOK // 2), jnp.int32),
    )
    q, vq, commit, cbl, ent, idx = pl.pallas_call(
        _vq_body,
        out_shape=out_shape,
    )(x, codebook)
    return (q,
            vq.reshape(()), commit.reshape(()), cbl.reshape(()),
            ent.reshape(()),
            idx)
